# Initial kernel scaffold; baseline (speedup 1.0000x reference)
import jax
import jax.numpy as jnp
from jax.experimental import pallas as pl


def kernel(x, W):
    raise NotImplementedError("write your pallas kernel here")


# speedup vs baseline: 2.6401x; 2.6401x over previous
import jax
import jax.numpy as jnp
from jax.experimental import pallas as pl

BETA = 0.25


def _vq_loss_kernel(x_ref, w_ref, out_ref, *, scale):
    i = pl.program_id(0)
    last = pl.num_programs(0) - 1
    xb = x_ref[0]
    w = w_ref[...]
    wsq = jnp.sum(w * w, axis=1, keepdims=True)
    scores = jax.lax.dot_general(
        w.astype(jnp.bfloat16), xb.astype(jnp.bfloat16),
        dimension_numbers=(((1,), (0,)), ((), ())),
        preferred_element_type=jnp.float32)
    dmin = jnp.min(wsq - 2.0 * scores, axis=0)
    xsq = jnp.sum(xb * xb, axis=0)
    partial = jnp.sum(dmin + xsq)
    total = jnp.where(i == 0, 0.0, out_ref[0, 0]) + partial
    out_ref[...] = jnp.where(i == last, total * scale, total).reshape(1, 1)


def kernel(x, W):
    b, c, h, w = x.shape
    pos = h * w
    codes, dim = W.shape
    xr = x.reshape(b, c, pos)
    scale = (1.0 + BETA) / float(x.size)
    import functools
    body = functools.partial(_vq_loss_kernel, scale=scale)
    loss = pl.pallas_call(
        body,
        grid=(b,),
        in_specs=[
            pl.BlockSpec((1, c, pos), lambda i: (i, 0, 0)),
            pl.BlockSpec((codes, dim), lambda i: (0, 0)),
        ],
        out_specs=pl.BlockSpec((1, 1), lambda i: (0, 0)),
        out_shape=jax.ShapeDtypeStruct((1, 1), jnp.float32),
    )(xr, W)[0, 0]
    return (x, loss)
